# baseline (speedup 1.0000x reference)
import functools

import jax
import jax.numpy as jnp
from jax import lax
from jax.experimental import pallas as pl
from jax.experimental.pallas import tpu as pltpu
from jax.experimental.pallas import tpu_sc as plsc

N_NODES = 10000
D = 128
NUM_CORES = 2
NUM_SUBCORES = 16
NW = NUM_CORES * NUM_SUBCORES
CH = 128
GC = 20
ROWS_PER_TILE = 640
N_ACC = NUM_SUBCORES * ROWS_PER_TILE


def _sc_aggregate(groups_per_worker, n_groups):
    mesh = plsc.VectorSubcoreMesh(core_axis_name="c", subcore_axis_name="s")

    @functools.partial(
        pl.kernel,
        mesh=mesh,
        out_type=jax.ShapeDtypeStruct((NUM_CORES * N_ACC, D), jnp.float32),
        scratch_types=[
            pltpu.VMEM((GC, CH), jnp.int32),
            pltpu.VMEM((GC, CH), jnp.int32),
            pltpu.VMEM((GC, CH), jnp.int32),
            pltpu.VMEM((GC, CH), jnp.int32),
            pltpu.VMEM((CH, D), jnp.float32),
            pltpu.VMEM((CH, D), jnp.float32),
            pltpu.VMEM_SHARED((N_ACC, D), jnp.float32),
            pltpu.SemaphoreType.DMA,
            pltpu.SemaphoreType.DMA,
            pltpu.SemaphoreType.DMA,
            pltpu.SemaphoreType.DMA,
            pltpu.SemaphoreType.DMA,
            pltpu.SemaphoreType.DMA,
        ],
    )
    def agg(x_hbm, src_hbm, dst_hbm, out_hbm, src_a, dst_a, src_b, dst_b,
            rows_a, rows_b, acc_sh, sem_a, sem_b, isem_a, isem_b, ssem_a,
            ssem_b):
        c = lax.axis_index("c")
        s = lax.axis_index("s")
        wid = c * NUM_SUBCORES + s

        z = jnp.zeros((16,), jnp.float32)

        def zrow(i, carry):
            for j in range(D // 16):
                rows_a[i, pl.ds(j * 16, 16)] = z
            return carry

        lax.fori_loop(0, CH, zrow, 0)
        for k in range(ROWS_PER_TILE // CH):
            pltpu.sync_copy(
                rows_a, acc_sh.at[pl.ds(s * ROWS_PER_TILE + k * CH, CH)])
        plsc.subcore_barrier()

        base_group = wid * groups_per_worker

        def load_group(g, srcv, dstv, isem):
            gi = jnp.minimum(base_group + g, n_groups - 1)
            pltpu.async_copy(src_hbm.at[gi], srcv, isem)
            pltpu.async_copy(dst_hbm.at[gi], dstv, isem)

        def wait_idx(srcv, dstv, isem):
            pltpu.make_async_copy(src_hbm.at[0], srcv, isem).wait()
            pltpu.make_async_copy(src_hbm.at[0], dstv, isem).wait()

        def wait_rows(buf, sem):
            pltpu.make_async_copy(x_hbm.at[pl.ds(0, CH)], buf, sem).wait()

        def run_group(g, srcv, dstv):
            @pl.when(base_group + g < n_groups)
            def _():
                pltpu.async_copy(x_hbm.at[srcv.at[0]], rows_a, sem_a)
                for j in range(GC // 2):
                    pltpu.async_copy(x_hbm.at[srcv.at[2 * j + 1]], rows_b,
                                     sem_b)
                    wait_rows(rows_a, sem_a)
                    pltpu.async_copy(rows_a, acc_sh.at[dstv.at[2 * j]],
                                     ssem_a, add=True)
                    wait_rows(rows_b, sem_b)
                    pltpu.async_copy(rows_b, acc_sh.at[dstv.at[2 * j + 1]],
                                     ssem_b, add=True)
                    wait_rows(rows_a, ssem_a)
                    if j < GC // 2 - 1:
                        pltpu.async_copy(x_hbm.at[srcv.at[2 * j + 2]],
                                         rows_a, sem_a)
                    wait_rows(rows_b, ssem_b)

        load_group(0, src_a, dst_a, isem_a)

        def outer(k, carry):
            g0 = 2 * k
            load_group(g0 + 1, src_b, dst_b, isem_b)
            wait_idx(src_a, dst_a, isem_a)
            run_group(g0, src_a, dst_a)

            @pl.when(g0 + 2 < groups_per_worker)
            def _():
                load_group(g0 + 2, src_a, dst_a, isem_a)
            wait_idx(src_b, dst_b, isem_b)
            run_group(g0 + 1, src_b, dst_b)
            return carry

        lax.fori_loop(0, groups_per_worker // 2, outer, 0)
        if groups_per_worker % 2:
            wait_idx(src_a, dst_a, isem_a)
            run_group(groups_per_worker - 1, src_a, dst_a)

        plsc.subcore_barrier()
        pltpu.sync_copy(
            acc_sh.at[pl.ds(s * ROWS_PER_TILE, ROWS_PER_TILE)],
            out_hbm.at[pl.ds(c * N_ACC + s * ROWS_PER_TILE, ROWS_PER_TILE)])

    return agg


def _mm_body(p0_ref, p1_ref, w_ref, b_ref, o_ref):
    agg = p0_ref[0] + p1_ref[0]
    o_ref[...] = jnp.dot(agg, w_ref[...],
                         preferred_element_type=jnp.float32) + b_ref[...]


def kernel(x, edge_index, weight, bias):
    n, d = x.shape
    e = edge_index.shape[1]
    src = edge_index[0]
    dst = edge_index[1]

    eg = GC * CH
    n_groups = -(-e // eg)
    pad = n_groups * eg - e
    if pad:
        r = jnp.arange(pad, dtype=jnp.int32)
        src = jnp.concatenate([src, r % n])
        dst = jnp.concatenate([dst, n + r % (N_ACC - n)])
    groups_per_worker = -(-n_groups // NW)

    src3 = src.reshape(n_groups, GC, CH)
    dst3 = dst.reshape(n_groups, GC, CH)
    partials = _sc_aggregate(groups_per_worker, n_groups)(x, src3, dst3)
    partials = partials.reshape(NUM_CORES, N_ACC, D)

    bm = 2000
    out = pl.pallas_call(
        _mm_body,
        grid=(n // bm,),
        in_specs=[
            pl.BlockSpec((1, bm, D), lambda i: (0, i, 0)),
            pl.BlockSpec((1, bm, D), lambda i: (1, i, 0)),
            pl.BlockSpec((D, D), lambda i: (0, 0)),
            pl.BlockSpec((1, D), lambda i: (0, 0)),
        ],
        out_specs=pl.BlockSpec((bm, D), lambda i: (i, 0)),
        out_shape=jax.ShapeDtypeStruct((n, D), jnp.float32),
    )(partials, partials, weight, bias.reshape(1, D))
    return out


# speedup vs baseline: 1.2236x; 1.2236x over previous
import functools

import jax
import jax.numpy as jnp
from jax import lax
from jax.experimental import pallas as pl
from jax.experimental.pallas import tpu as pltpu
from jax.experimental.pallas import tpu_sc as plsc

N_NODES = 10000
D = 128
NUM_CORES = 2
NUM_SUBCORES = 16
NW = NUM_CORES * NUM_SUBCORES
CH = 128
GC = 20
ROWS_PER_TILE = 640
N_ACC = NUM_SUBCORES * ROWS_PER_TILE


def _sc_aggregate(groups_per_worker, n_groups):
    mesh = plsc.VectorSubcoreMesh(core_axis_name="c", subcore_axis_name="s")

    @functools.partial(
        pl.kernel,
        mesh=mesh,
        out_type=jax.ShapeDtypeStruct((NUM_CORES * N_ACC, D), jnp.float32),
        scratch_types=[
            pltpu.VMEM((GC, CH), jnp.int32),
            pltpu.VMEM((GC, CH), jnp.int32),
            pltpu.VMEM((GC, CH), jnp.int32),
            pltpu.VMEM((GC, CH), jnp.int32),
            pltpu.VMEM((CH, D), jnp.float32),
            pltpu.VMEM((CH, D), jnp.float32),
            pltpu.VMEM_SHARED((N_ACC, D), jnp.float32),
            pltpu.SemaphoreType.DMA,
            pltpu.SemaphoreType.DMA,
            pltpu.SemaphoreType.DMA,
            pltpu.SemaphoreType.DMA,
        ],
    )
    def agg(x_hbm, src_hbm, dst_hbm, out_hbm, src_a, dst_a, src_b, dst_b,
            rows_a, rows_b, acc_sh, sem_a, sem_b, isem_a, isem_b):
        c = lax.axis_index("c")
        s = lax.axis_index("s")
        wid = c * NUM_SUBCORES + s
        base_group = wid * groups_per_worker

        def load_group(g, srcv, dstv, isem):
            gi = jnp.minimum(base_group + g, n_groups - 1)
            pltpu.async_copy(src_hbm.at[gi], srcv, isem)
            pltpu.async_copy(dst_hbm.at[gi], dstv, isem)

        load_group(0, src_a, dst_a, isem_a)

        z = jnp.zeros((16,), jnp.float32)

        def zrow(i, carry):
            for j in range(D // 16):
                rows_a[i, pl.ds(j * 16, 16)] = z
            return carry

        lax.fori_loop(0, CH, zrow, 0)
        for k in range(ROWS_PER_TILE // CH):
            pltpu.sync_copy(
                rows_a, acc_sh.at[pl.ds(s * ROWS_PER_TILE + k * CH, CH)])
        plsc.subcore_barrier()

        def wait_idx(srcv, dstv, isem):
            pltpu.make_async_copy(src_hbm.at[0], srcv, isem).wait()
            pltpu.make_async_copy(src_hbm.at[0], dstv, isem).wait()

        def wait_rows(buf, sem):
            pltpu.make_async_copy(x_hbm.at[pl.ds(0, CH)], buf, sem).wait()

        def run_group(g, srcv, dstv):
            @pl.when(base_group + g < n_groups)
            def _():
                pltpu.async_copy(x_hbm.at[srcv.at[0]], rows_a, sem_a)
                for j in range(GC // 2):
                    pltpu.async_copy(x_hbm.at[srcv.at[2 * j + 1]], rows_b,
                                     sem_b)
                    wait_rows(rows_a, sem_a)
                    pltpu.sync_copy(rows_a, acc_sh.at[dstv.at[2 * j]],
                                    add=True)
                    if j < GC // 2 - 1:
                        pltpu.async_copy(x_hbm.at[srcv.at[2 * j + 2]],
                                         rows_a, sem_a)
                    wait_rows(rows_b, sem_b)
                    pltpu.sync_copy(rows_b, acc_sh.at[dstv.at[2 * j + 1]],
                                    add=True)

        def outer(k, carry):
            g0 = 2 * k
            load_group(g0 + 1, src_b, dst_b, isem_b)
            wait_idx(src_a, dst_a, isem_a)
            run_group(g0, src_a, dst_a)

            @pl.when(g0 + 2 < groups_per_worker)
            def _():
                load_group(g0 + 2, src_a, dst_a, isem_a)
            wait_idx(src_b, dst_b, isem_b)
            run_group(g0 + 1, src_b, dst_b)
            return carry

        lax.fori_loop(0, groups_per_worker // 2, outer, 0)
        if groups_per_worker % 2:
            wait_idx(src_a, dst_a, isem_a)
            run_group(groups_per_worker - 1, src_a, dst_a)

        plsc.subcore_barrier()
        pltpu.sync_copy(
            acc_sh.at[pl.ds(s * ROWS_PER_TILE, ROWS_PER_TILE)],
            out_hbm.at[pl.ds(c * N_ACC + s * ROWS_PER_TILE, ROWS_PER_TILE)])

    return agg


def _mm_body(p0_ref, p1_ref, w_ref, b_ref, o_ref):
    agg = p0_ref[0] + p1_ref[0]
    o_ref[...] = jnp.dot(agg, w_ref[...],
                         preferred_element_type=jnp.float32) + b_ref[...]


def kernel(x, edge_index, weight, bias):
    n, d = x.shape
    e = edge_index.shape[1]
    src = edge_index[0]
    dst = edge_index[1]

    eg = GC * CH
    n_groups = -(-e // eg)
    pad = n_groups * eg - e
    if pad:
        r = jnp.arange(pad, dtype=jnp.int32)
        src = jnp.concatenate([src, r % n])
        dst = jnp.concatenate([dst, n + r % (N_ACC - n)])
    groups_per_worker = -(-n_groups // NW)

    src3 = src.reshape(n_groups, GC, CH)
    dst3 = dst.reshape(n_groups, GC, CH)
    partials = _sc_aggregate(groups_per_worker, n_groups)(x, src3, dst3)
    partials = partials.reshape(NUM_CORES, N_ACC, D)

    bm = 2000
    out = pl.pallas_call(
        _mm_body,
        grid=(n // bm,),
        in_specs=[
            pl.BlockSpec((1, bm, D), lambda i: (0, i, 0)),
            pl.BlockSpec((1, bm, D), lambda i: (1, i, 0)),
            pl.BlockSpec((D, D), lambda i: (0, 0)),
            pl.BlockSpec((1, D), lambda i: (0, 0)),
        ],
        out_specs=pl.BlockSpec((bm, D), lambda i: (i, 0)),
        out_shape=jax.ShapeDtypeStruct((n, D), jnp.float32),
    )(partials, partials, weight, bias.reshape(1, D))
    return out


# speedup vs baseline: 1.2284x; 1.0040x over previous
import functools

import jax
import jax.numpy as jnp
from jax import lax
from jax.experimental import pallas as pl
from jax.experimental.pallas import tpu as pltpu
from jax.experimental.pallas import tpu_sc as plsc

N_NODES = 10000
D = 128
NUM_CORES = 2
NUM_SUBCORES = 16
NW = NUM_CORES * NUM_SUBCORES
CH = 128
GC = 20
ROWS_PER_TILE = 640
N_ACC = NUM_SUBCORES * ROWS_PER_TILE


def _sc_aggregate(groups_per_worker, n_groups):
    mesh = plsc.VectorSubcoreMesh(core_axis_name="c", subcore_axis_name="s")

    @functools.partial(
        pl.kernel,
        mesh=mesh,
        out_type=jax.ShapeDtypeStruct((NUM_CORES * N_ACC, D), jnp.float32),
        scratch_types=[
            pltpu.VMEM((GC, CH), jnp.int32),
            pltpu.VMEM((GC, CH), jnp.int32),
            pltpu.VMEM((GC, CH), jnp.int32),
            pltpu.VMEM((GC, CH), jnp.int32),
            pltpu.VMEM((CH, D), jnp.float32),
            pltpu.VMEM((CH, D), jnp.float32),
            pltpu.VMEM_SHARED((N_ACC, D), jnp.float32),
            pltpu.SemaphoreType.DMA,
            pltpu.SemaphoreType.DMA,
            pltpu.SemaphoreType.DMA,
            pltpu.SemaphoreType.DMA,
        ],
    )
    def agg(x_hbm, src_hbm, dst_hbm, out_hbm, src_a, dst_a, src_b, dst_b,
            rows_a, rows_b, acc_sh, sem_a, sem_b, isem_a, isem_b):
        c = lax.axis_index("c")
        s = lax.axis_index("s")
        wid = c * NUM_SUBCORES + s
        base_group = wid * groups_per_worker

        def load_group(g, srcv, dstv, isem):
            gi = jnp.minimum(base_group + g, n_groups - 1)
            pltpu.async_copy(src_hbm.at[gi], srcv, isem)
            pltpu.async_copy(dst_hbm.at[gi], dstv, isem)

        load_group(0, src_a, dst_a, isem_a)

        z = jnp.zeros((16,), jnp.float32)

        def zrow(i, carry):
            for j in range(D // 16):
                rows_a[i, pl.ds(j * 16, 16)] = z
            return carry

        lax.fori_loop(0, CH, zrow, 0)
        for k in range(ROWS_PER_TILE // CH):
            pltpu.sync_copy(
                rows_a, acc_sh.at[pl.ds(s * ROWS_PER_TILE + k * CH, CH)])
        plsc.subcore_barrier()

        def wait_idx(srcv, dstv, isem):
            pltpu.make_async_copy(src_hbm.at[0], srcv, isem).wait()
            pltpu.make_async_copy(src_hbm.at[0], dstv, isem).wait()

        def wait_rows(buf, sem):
            pltpu.make_async_copy(x_hbm.at[pl.ds(0, CH)], buf, sem).wait()

        def prime(g, srcv):
            @pl.when(base_group + g < n_groups)
            def _():
                pltpu.async_copy(x_hbm.at[srcv.at[0]], rows_a, sem_a)

        def run_group(g, srcv, dstv):
            @pl.when(base_group + g < n_groups)
            def _():
                for j in range(GC // 2):
                    pltpu.async_copy(x_hbm.at[srcv.at[2 * j + 1]], rows_b,
                                     sem_b)
                    wait_rows(rows_a, sem_a)
                    pltpu.sync_copy(rows_a, acc_sh.at[dstv.at[2 * j]],
                                    add=True)
                    if j < GC // 2 - 1:
                        pltpu.async_copy(x_hbm.at[srcv.at[2 * j + 2]],
                                         rows_a, sem_a)
                    wait_rows(rows_b, sem_b)
                    pltpu.sync_copy(rows_b, acc_sh.at[dstv.at[2 * j + 1]],
                                    add=True)

        assert groups_per_worker % 2 == 0
        wait_idx(src_a, dst_a, isem_a)
        prime(0, src_a)
        load_group(1, src_b, dst_b, isem_b)

        def outer(k, carry):
            g0 = 2 * k
            run_group(g0, src_a, dst_a)
            wait_idx(src_b, dst_b, isem_b)
            prime(g0 + 1, src_b)

            @pl.when(g0 + 2 < groups_per_worker)
            def _():
                load_group(g0 + 2, src_a, dst_a, isem_a)
            run_group(g0 + 1, src_b, dst_b)

            @pl.when(g0 + 2 < groups_per_worker)
            def _():
                wait_idx(src_a, dst_a, isem_a)
                prime(g0 + 2, src_a)

            @pl.when(g0 + 3 < groups_per_worker)
            def _():
                load_group(g0 + 3, src_b, dst_b, isem_b)
            return carry

        lax.fori_loop(0, groups_per_worker // 2, outer, 0)

        plsc.subcore_barrier()
        pltpu.sync_copy(
            acc_sh.at[pl.ds(s * ROWS_PER_TILE, ROWS_PER_TILE)],
            out_hbm.at[pl.ds(c * N_ACC + s * ROWS_PER_TILE, ROWS_PER_TILE)])

    return agg


def _mm_body(p0_ref, p1_ref, w_ref, b_ref, o_ref):
    agg = p0_ref[0] + p1_ref[0]
    o_ref[...] = jnp.dot(agg, w_ref[...],
                         preferred_element_type=jnp.float32) + b_ref[...]


def kernel(x, edge_index, weight, bias):
    n, d = x.shape
    e = edge_index.shape[1]
    src = edge_index[0]
    dst = edge_index[1]

    eg = GC * CH
    n_groups = -(-e // eg)
    pad = n_groups * eg - e
    if pad:
        r = jnp.arange(pad, dtype=jnp.int32)
        src = jnp.concatenate([src, r % n])
        dst = jnp.concatenate([dst, n + r % (N_ACC - n)])
    groups_per_worker = -(-n_groups // NW)
    groups_per_worker += groups_per_worker % 2

    src3 = src.reshape(n_groups, GC, CH)
    dst3 = dst.reshape(n_groups, GC, CH)
    partials = _sc_aggregate(groups_per_worker, n_groups)(x, src3, dst3)
    partials = partials.reshape(NUM_CORES, N_ACC, D)

    bm = 2000
    out = pl.pallas_call(
        _mm_body,
        grid=(n // bm,),
        in_specs=[
            pl.BlockSpec((1, bm, D), lambda i: (0, i, 0)),
            pl.BlockSpec((1, bm, D), lambda i: (1, i, 0)),
            pl.BlockSpec((D, D), lambda i: (0, 0)),
            pl.BlockSpec((1, D), lambda i: (0, 0)),
        ],
        out_specs=pl.BlockSpec((bm, D), lambda i: (i, 0)),
        out_shape=jax.ShapeDtypeStruct((n, D), jnp.float32),
    )(partials, partials, weight, bias.reshape(1, D))
    return out
